# baseline (speedup 1.0000x reference)
import jax
import jax.numpy as jnp
from jax import lax
from jax.experimental import pallas as pl
from jax.experimental.pallas import tpu as pltpu
import jax.experimental.pallas.tpu_sc as plsc

N = 5000
NPAD = 5120
NUM_CLASSES = 21
C = NUM_CLASSES - 1
NTOT = C * NPAD
BLK = 64
NBLK_C = NPAD // BLK
NBLK = NTOT // BLK
NUM_DET = 100
SCORE_T = 0.05
NMS_T = 0.5
IMG_W = 1333.0
IMG_H = 800.0
MIN_SIZE = 1.0
LOG_MAX = 4.135166556742356
OFF_STEP = IMG_W + 2.0
NEG_INF = float("-inf")

_INTERPRET = False


def _precompute_body(logit_ref, d4_ref, prop_ref, nbx1_ref, nby1_ref,
                     nbx2_ref, nby2_ref, work_ref,
                     rx1_ref, ry1_ref, rx2_ref, ry2_ref, s0_ref):
    logit = logit_ref[...]
    m = jnp.max(logit, axis=0, keepdims=True)
    e = jnp.exp(logit - m)
    ssum = jnp.sum(e, axis=0, keepdims=True)
    scores_all = e / ssum
    scores = scores_all[1:, :]

    px1 = prop_ref[0, :][None, :]
    py1 = prop_ref[1, :][None, :]
    px2 = prop_ref[2, :][None, :]
    py2 = prop_ref[3, :][None, :]
    widths = px2 - px1
    heights = py2 - py1
    ctr_x = px1 + 0.5 * widths
    ctr_y = py1 + 0.5 * heights

    dx = d4_ref[0] / 10.0
    dy = d4_ref[1] / 10.0
    dw = jnp.minimum(d4_ref[2] / 5.0, LOG_MAX)
    dh = jnp.minimum(d4_ref[3] / 5.0, LOG_MAX)

    pcx = dx * widths + ctr_x
    pcy = dy * heights + ctr_y
    pw = jnp.exp(dw) * widths
    ph = jnp.exp(dh) * heights

    x1 = jnp.clip(pcx - 0.5 * pw, 0.0, IMG_W)
    y1 = jnp.clip(pcy - 0.5 * ph, 0.0, IMG_H)
    x2 = jnp.clip(pcx + 0.5 * pw, 0.0, IMG_W)
    y2 = jnp.clip(pcy + 0.5 * ph, 0.0, IMG_H)

    cls_iota = lax.broadcasted_iota(jnp.int32, (C, NPAD), 0).astype(jnp.float32)
    offset = (cls_iota + 1.0) * OFF_STEP
    nbx1 = x1 + offset
    nby1 = y1 + offset
    nbx2 = x2 + offset
    nby2 = y2 + offset

    ws = x2 - x1
    hs = y2 - y1
    lane = lax.broadcasted_iota(jnp.int32, (C, NPAD), 1)
    valid = (scores > SCORE_T) & (ws >= MIN_SIZE) & (hs >= MIN_SIZE) \
        & (lane < N)
    work = jnp.where(valid, scores, NEG_INF)

    nbx1_ref[...] = nbx1
    nby1_ref[...] = nby1
    nbx2_ref[...] = nbx2
    nby2_ref[...] = nby2
    work_ref[...] = work
    rx1_ref[...] = x1
    ry1_ref[...] = y1
    rx2_ref[...] = x2
    ry2_ref[...] = y2
    s0_ref[...] = scores[0:1, 0:1]


def _bmax_body(w_ref, bmax_ref):
    bmax_ref[...] = jnp.max(w_ref[...], axis=1)


def _sc_nms(work_flat, bmax, rec, s0):
    mesh = plsc.VectorSubcoreMesh(core_axis_name="c", subcore_axis_name="s",
                                  num_cores=2, num_subcores=16)

    def body(work_hbm, bmax_hbm, rec_hbm, s0_hbm, out_hbm,
             rec_sh, work_sh, bmax_v, clsmax_v, selx1_v, sely1_v, selx2_v,
             sely2_v, sela_v, sellb_v, rec_v, s0_v, out_v, wblk_v,
             sem_rec, sem_a, sem_b):
        cid = lax.axis_index("c")
        sid = lax.axis_index("s")

        @pl.when((cid == 0) & (sid == 0))
        def _():
            i16 = lax.broadcasted_iota(jnp.int32, (16,), 0)

            rec_copy = pltpu.make_async_copy(rec_hbm, rec_sh, sem_rec)
            rec_copy.start()
            pltpu.async_copy(work_hbm, work_sh, sem_a).wait()
            pltpu.async_copy(bmax_hbm, bmax_v, sem_b).wait()
            pltpu.async_copy(s0_hbm, s0_v.at[pl.ds(0, 8)], sem_b).wait()

            cm0 = jnp.full((16,), NEG_INF, jnp.float32)
            cm1 = jnp.full((16,), NEG_INF, jnp.float32)
            for c in range(C):
                mv = jnp.full((16,), NEG_INF, jnp.float32)
                for j in range(NBLK_C // 16):
                    mv = jnp.maximum(mv, bmax_v[pl.ds(c * NBLK_C + 16 * j,
                                                      16)])
                mc = jnp.max(mv)
                if c < 16:
                    cm0 = jnp.where(i16 == c, mc, cm0)
                else:
                    cm1 = jnp.where(i16 == (c - 16), mc, cm1)
            clsmax_v[pl.ds(0, 16)] = cm0
            clsmax_v[pl.ds(16, 16)] = cm1

            rec_copy.wait()
            s0 = s0_v[...][0]

            def nms_step(t, t_sel):
                def pop_cond(st):
                    return ~st[0]

                def pop_body(st):
                    ts = st[5]
                    cm0 = clsmax_v[pl.ds(0, 16)]
                    cm1 = clsmax_v[pl.ds(16, 16)]
                    v = jnp.maximum(jnp.max(cm0), jnp.max(cm1))
                    is_fb = v == NEG_INF
                    c0 = jnp.min(jnp.where(cm0 == v, i16, 999))
                    c1 = jnp.min(jnp.where(cm1 == v, i16 + 16, 999))
                    c = jnp.minimum(c0, c1)
                    c = jnp.where(is_fb, 0, c)

                    cbase = c * NBLK_C
                    bm = bmax_v[pl.ds(cbase, 16)]
                    bj = jnp.zeros((16,), jnp.int32)
                    for j in range(1, NBLK_C // 16):
                        ch = bmax_v[pl.ds(cbase + 16 * j, 16)]
                        gt = ch > bm
                        bm = jnp.where(gt, ch, bm)
                        bj = jnp.where(gt, j, bj)
                    k = jnp.min(jnp.where(bm == v, bj * 16 + i16, 9999))
                    k = jnp.where(is_fb, 0, k)

                    ebase = c * NPAD + k * BLK
                    pltpu.sync_copy(work_sh.at[pl.ds(ebase, BLK)], wblk_v)
                    l = jnp.int32(9999)
                    for j in range(BLK // 16):
                        ch = wblk_v[pl.ds(16 * j, 16)]
                        lj = jnp.min(jnp.where(ch == v, i16 + 16 * j, 9999))
                        l = jnp.minimum(l, lj)
                    l = jnp.where(is_fb, 0, l)
                    i = ebase + l

                    pltpu.sync_copy(rec_sh.at[pl.ds((i // 2) * 8, 8)],
                                    rec_v.at[pl.ds(0, 8)])
                    rv = rec_v[...]
                    podd = (i % 2) == 1
                    bx1 = jnp.where(podd, rv[4], rv[0])
                    by1 = jnp.where(podd, rv[5], rv[1])
                    bx2 = jnp.where(podd, rv[6], rv[2])
                    by2 = jnp.where(podd, rv[7], rv[3])
                    ca = (bx2 - bx1) * (by2 - by1)
                    cf = c.astype(jnp.float32)

                    nq = (ts + 15) // 16

                    def tbody(q, any_v):
                        qb = 16 * q
                        sx1 = selx1_v[pl.ds(qb, 16)]
                        sy1 = sely1_v[pl.ds(qb, 16)]
                        sx2 = selx2_v[pl.ds(qb, 16)]
                        sy2 = sely2_v[pl.ds(qb, 16)]
                        sa = sela_v[pl.ds(qb, 16)]
                        sl = sellb_v[pl.ds(qb, 16)]
                        xx1 = jnp.maximum(sx1, bx1)
                        yy1 = jnp.maximum(sy1, by1)
                        xx2 = jnp.minimum(sx2, bx2)
                        yy2 = jnp.minimum(sy2, by2)
                        inter = (jnp.maximum(xx2 - xx1, 0.0)
                                 * jnp.maximum(yy2 - yy1, 0.0))
                        iou = inter / (ca + sa - inter + 1e-9)
                        mask = ((sl == cf) & ((qb + i16) < ts)
                                & (iou > NMS_T))
                        return any_v | mask

                    viol = lax.fori_loop(0, nq, tbody,
                                         jnp.zeros((16,), jnp.bool_))
                    suppressed = jnp.any(viol) & ~is_fb

                    @pl.when(~is_fb)
                    def _():
                        chb = (l // 16) * 16
                        ch = wblk_v[pl.ds(chb, 16)]
                        wblk_v[pl.ds(chb, 16)] = jnp.where(
                            i16 == (l % 16), NEG_INF, ch)
                        pltpu.sync_copy(wblk_v, work_sh.at[pl.ds(ebase, BLK)])
                        nm = jnp.full((16,), NEG_INF, jnp.float32)
                        for j in range(BLK // 16):
                            nm = jnp.maximum(
                                nm, wblk_v[pl.ds(16 * j, 16)])
                        nbm = jnp.max(nm)
                        kb = cbase + (k // 16) * 16
                        bch = bmax_v[pl.ds(kb, 16)]
                        bmax_v[pl.ds(kb, 16)] = jnp.where(
                            i16 == (k % 16), nbm, bch)
                        cmv = jnp.full((16,), NEG_INF, jnp.float32)
                        for j in range(NBLK_C // 16):
                            cmv = jnp.maximum(
                                cmv, bmax_v[pl.ds(cbase + 16 * j, 16)])
                        newcm = jnp.max(cmv)
                        m0 = clsmax_v[pl.ds(0, 16)]
                        m1 = clsmax_v[pl.ds(16, 16)]
                        clsmax_v[pl.ds(0, 16)] = jnp.where(
                            i16 == c, newcm, m0)
                        clsmax_v[pl.ds(16, 16)] = jnp.where(
                            i16 == (c - 16), newcm, m1)

                    return (~suppressed, c, v, is_fb, i, ts)

                done, c, v, is_fb, i_acc, t_sel = lax.while_loop(
                    pop_cond, pop_body,
                    (jnp.bool_(False), jnp.int32(0), jnp.float32(0),
                     jnp.bool_(False), jnp.int32(0), t_sel))

                rv = rec_v[...]
                podd = (i_acc % 2) == 1
                bx1 = jnp.where(podd, rv[4], rv[0])
                by1 = jnp.where(podd, rv[5], rv[1])
                bx2 = jnp.where(podd, rv[6], rv[2])
                by2 = jnp.where(podd, rv[7], rv[3])
                ca = (bx2 - bx1) * (by2 - by1)
                cf = c.astype(jnp.float32)
                offc = (cf + 1.0) * OFF_STEP
                rx1 = bx1 - offc
                ry1 = by1 - offc
                rx2 = bx2 - offc
                ry2 = by2 - offc

                @pl.when(~is_fb)
                def _():
                    qb = (t_sel // 16) * 16
                    lq = t_sel % 16
                    selx1_v[pl.ds(qb, 16)] = jnp.where(
                        i16 == lq, bx1, selx1_v[pl.ds(qb, 16)])
                    sely1_v[pl.ds(qb, 16)] = jnp.where(
                        i16 == lq, by1, sely1_v[pl.ds(qb, 16)])
                    selx2_v[pl.ds(qb, 16)] = jnp.where(
                        i16 == lq, bx2, selx2_v[pl.ds(qb, 16)])
                    sely2_v[pl.ds(qb, 16)] = jnp.where(
                        i16 == lq, by2, sely2_v[pl.ds(qb, 16)])
                    sela_v[pl.ds(qb, 16)] = jnp.where(
                        i16 == lq, ca, sela_v[pl.ds(qb, 16)])
                    sellb_v[pl.ds(qb, 16)] = jnp.where(
                        i16 == lq, cf, sellb_v[pl.ds(qb, 16)])

                score = jnp.where(is_fb, s0, v)
                lbl = cf + 1.0
                row = jnp.where(
                    i16 == 0, rx1,
                    jnp.where(i16 == 1, ry1,
                              jnp.where(i16 == 2, rx2,
                                        jnp.where(i16 == 3, ry2,
                                                  jnp.where(i16 == 4, score,
                                                            lbl)))))
                out_v[pl.ds(t * 16, 16)] = row
                return jnp.where(is_fb, t_sel, t_sel + 1)

            lax.fori_loop(0, NUM_DET, nms_step, jnp.int32(0))
            pltpu.sync_copy(out_v, out_hbm)

    f = pl.kernel(
        body,
        out_type=jax.ShapeDtypeStruct((NUM_DET * 16,), jnp.float32),
        mesh=mesh,
        compiler_params=pltpu.CompilerParams(needs_layout_passes=False),
        scratch_types=[
            pltpu.VMEM_SHARED((NTOT * 4,), jnp.float32),
            pltpu.VMEM_SHARED((NTOT,), jnp.float32),
            pltpu.VMEM((NBLK,), jnp.float32),
            pltpu.VMEM((32,), jnp.float32),
            pltpu.VMEM((128,), jnp.float32),
            pltpu.VMEM((128,), jnp.float32),
            pltpu.VMEM((128,), jnp.float32),
            pltpu.VMEM((128,), jnp.float32),
            pltpu.VMEM((128,), jnp.float32),
            pltpu.VMEM((128,), jnp.float32),
            pltpu.VMEM((16,), jnp.float32),
            pltpu.VMEM((16,), jnp.float32),
            pltpu.VMEM((NUM_DET * 16,), jnp.float32),
            pltpu.VMEM((BLK,), jnp.float32),
            pltpu.SemaphoreType.DMA,
            pltpu.SemaphoreType.DMA,
            pltpu.SemaphoreType.DMA,
        ],
        interpret=_INTERPRET,
    )
    return f(work_flat, bmax, rec, s0)


@jax.jit
def kernel(class_logit, box_regression, proposal):
    logit_t = jnp.pad(class_logit, ((0, NPAD - N), (0, 0))).T
    d4 = jnp.pad(
        jnp.transpose(box_regression.reshape(N, NUM_CLASSES, 4)[:, 1:, :],
                      (2, 1, 0)),
        ((0, 0), (0, 0), (0, NPAD - N)))
    prop_t = jnp.pad(proposal, ((0, NPAD - N), (0, 0))).T

    big = jax.ShapeDtypeStruct((C, NPAD), jnp.float32)
    pre = pl.pallas_call(
        _precompute_body,
        out_shape=(big,) * 9 + (jax.ShapeDtypeStruct((1, 1), jnp.float32),),
        interpret=_INTERPRET,
    )(logit_t, d4, prop_t)
    nbx1, nby1, nbx2, nby2, work, rx1, ry1, rx2, ry2, s0 = pre

    bmax = pl.pallas_call(
        _bmax_body,
        out_shape=jax.ShapeDtypeStruct((NBLK,), jnp.float32),
        interpret=_INTERPRET,
    )(work.reshape(NBLK, BLK))

    rec = jnp.stack([nbx1, nby1, nbx2, nby2], axis=-1).reshape(-1)
    s0v = jnp.broadcast_to(s0.reshape(1), (8,))

    out = _sc_nms(work.reshape(-1), bmax, rec, s0v).reshape(NUM_DET, 16)

    boxes = out[:, 0:4]
    scores = out[:, 4]
    labels = out[:, 5].astype(jnp.int32)
    return boxes, scores, labels


# speedup vs baseline: 1.9371x; 1.9371x over previous
import jax
import jax.numpy as jnp
from jax import lax
from jax.experimental import pallas as pl
from jax.experimental.pallas import tpu as pltpu

N = 5000
NPAD = 5120
NBLK = 8
NSUB = 640
NUM_CLASSES = 21
C = NUM_CLASSES - 1
NUM_DET = 100
SCORE_T = 0.05
NMS_T = 0.5
IMG_W = 1333.0
IMG_H = 800.0
MIN_SIZE = 1.0
LOG_MAX = 4.135166556742356
OFF_STEP = IMG_W + 2.0
NEG_INF = float("-inf")

_INTERPRET = False


def _precompute_body(logit_ref, d4_ref, prop_ref, nbx1_ref, nby1_ref,
                     nbx2_ref, nby2_ref, work_ref, s0_ref):
    logit = logit_ref[...]
    m = jnp.max(logit, axis=0, keepdims=True)
    e = jnp.exp(logit - m)
    ssum = jnp.sum(e, axis=0, keepdims=True)
    scores_all = e / ssum
    scores = scores_all[1:, :]

    px1 = prop_ref[0, :][None, :]
    py1 = prop_ref[1, :][None, :]
    px2 = prop_ref[2, :][None, :]
    py2 = prop_ref[3, :][None, :]
    widths = px2 - px1
    heights = py2 - py1
    ctr_x = px1 + 0.5 * widths
    ctr_y = py1 + 0.5 * heights

    dx = d4_ref[0] / 10.0
    dy = d4_ref[1] / 10.0
    dw = jnp.minimum(d4_ref[2] / 5.0, LOG_MAX)
    dh = jnp.minimum(d4_ref[3] / 5.0, LOG_MAX)

    pcx = dx * widths + ctr_x
    pcy = dy * heights + ctr_y
    pw = jnp.exp(dw) * widths
    ph = jnp.exp(dh) * heights

    x1 = jnp.clip(pcx - 0.5 * pw, 0.0, IMG_W)
    y1 = jnp.clip(pcy - 0.5 * ph, 0.0, IMG_H)
    x2 = jnp.clip(pcx + 0.5 * pw, 0.0, IMG_W)
    y2 = jnp.clip(pcy + 0.5 * ph, 0.0, IMG_H)

    cls_iota = lax.broadcasted_iota(jnp.int32, (C, NPAD), 0).astype(jnp.float32)
    offset = (cls_iota + 1.0) * OFF_STEP
    nbx1 = x1 + offset
    nby1 = y1 + offset
    nbx2 = x2 + offset
    nby2 = y2 + offset

    ws = x2 - x1
    hs = y2 - y1
    lane = lax.broadcasted_iota(jnp.int32, (C, NPAD), 1)
    valid = (scores > SCORE_T) & (ws >= MIN_SIZE) & (hs >= MIN_SIZE) \
        & (lane < N)
    work = jnp.where(valid, scores, NEG_INF)

    nbx1_ref[...] = nbx1
    nby1_ref[...] = nby1
    nbx2_ref[...] = nbx2
    nby2_ref[...] = nby2
    work_ref[...] = work
    s0_ref[...] = scores[0:1, 0:1]


def _nms_body(nbx1_ref, nby1_ref, nbx2_ref, nby2_ref, work_in_ref, s0_ref,
              out_ref, work_ref):
    work_ref[...] = work_in_ref[...]
    s0 = s0_ref[0, 0]

    lane128 = lax.broadcasted_iota(jnp.int32, (1, 128), 1)
    vec = jnp.full((1, 128), NEG_INF, dtype=jnp.float32)
    for c in range(C):
        mc = jnp.max(work_ref[c])
        vec = jnp.where(lane128 == c, mc, vec)

    sub_iota = lax.broadcasted_iota(jnp.int32, (NBLK, NSUB), 0)
    lane_iota = lax.broadcasted_iota(jnp.int32, (NBLK, NSUB), 1)
    flat_local = sub_iota * NSUB + lane_iota
    liota = lax.broadcasted_iota(jnp.int32, (1, 8), 1)

    def body(t, vec):
        v = jnp.max(vec)
        is_fb = v == NEG_INF
        cls = jnp.min(jnp.where(vec == v, lane128, 127))

        w_c = work_ref[cls]
        eq = (w_c == v) & (~is_fb | (flat_local == 0))

        nx1 = nbx1_ref[cls]
        ny1 = nby1_ref[cls]
        nx2 = nbx2_ref[cls]
        ny2 = nby2_ref[cls]
        bx1 = jnp.sum(jnp.where(eq, nx1, 0.0))
        by1 = jnp.sum(jnp.where(eq, ny1, 0.0))
        bx2 = jnp.sum(jnp.where(eq, nx2, 0.0))
        by2 = jnp.sum(jnp.where(eq, ny2, 0.0))
        ba = (bx2 - bx1) * (by2 - by1)

        xx1 = jnp.maximum(nx1, bx1)
        yy1 = jnp.maximum(ny1, by1)
        xx2 = jnp.minimum(nx2, bx2)
        yy2 = jnp.minimum(ny2, by2)
        areas = (nx2 - nx1) * (ny2 - ny1)
        inter = jnp.maximum(xx2 - xx1, 0.0) * jnp.maximum(yy2 - yy1, 0.0)
        iou = inter / (areas + ba - inter + 1e-9)
        new_w = jnp.where((iou > NMS_T) | eq, NEG_INF, w_c)
        work_ref[cls] = new_w

        mc = jnp.max(new_w)
        vec = jnp.where(lane128 == cls, mc, vec)

        s_out = jnp.where(is_fb, s0, v)
        offc = (cls + 1).astype(jnp.float32) * OFF_STEP
        lbl = (cls + 1).astype(jnp.float32)
        row = jnp.where(
            liota == 0, bx1 - offc,
            jnp.where(liota == 1, by1 - offc,
                      jnp.where(liota == 2, bx2 - offc,
                                jnp.where(liota == 3, by2 - offc,
                                          jnp.where(liota == 4, s_out, lbl)))))
        out_ref[pl.ds(t, 1), :] = row
        return vec

    lax.fori_loop(0, NUM_DET, body, vec)


@jax.jit
def kernel(class_logit, box_regression, proposal):
    logit_t = jnp.pad(class_logit, ((0, NPAD - N), (0, 0))).T
    d4 = jnp.pad(
        jnp.transpose(box_regression.reshape(N, NUM_CLASSES, 4)[:, 1:, :],
                      (2, 1, 0)),
        ((0, 0), (0, 0), (0, NPAD - N)))
    prop_t = jnp.pad(proposal, ((0, NPAD - N), (0, 0))).T

    big = jax.ShapeDtypeStruct((C, NPAD), jnp.float32)
    pre = pl.pallas_call(
        _precompute_body,
        out_shape=(big,) * 5 + (jax.ShapeDtypeStruct((1, 1), jnp.float32),),
        interpret=_INTERPRET,
    )(logit_t, d4, prop_t)
    nbx1, nby1, nbx2, nby2, work, s0 = pre

    shaped = [a.reshape(C, NBLK, NSUB)
              for a in (nbx1, nby1, nbx2, nby2, work)]

    out = pl.pallas_call(
        _nms_body,
        out_shape=jax.ShapeDtypeStruct((NUM_DET, 8), jnp.float32),
        scratch_shapes=[pltpu.VMEM((C, NBLK, NSUB), jnp.float32)],
        interpret=_INTERPRET,
    )(*shaped, s0)

    boxes = out[:, 0:4]
    scores = out[:, 4]
    labels = out[:, 5].astype(jnp.int32)
    return boxes, scores, labels


# speedup vs baseline: 2.7103x; 1.3992x over previous
import jax
import jax.numpy as jnp
from jax import lax
from jax.experimental import pallas as pl
from jax.experimental.pallas import tpu as pltpu

N = 5000
NPAD = 5120
NBLK = 8
NSUB = 640
NUM_CLASSES = 21
C = NUM_CLASSES - 1
NUM_DET = 100
SCORE_T = 0.05
NMS_T = 0.5
IMG_W = 1333.0
IMG_H = 800.0
MIN_SIZE = 1.0
LOG_MAX = 4.135166556742356
OFF_STEP = IMG_W + 2.0
NEG_INF = float("-inf")

_INTERPRET = False


def _precompute_body(logit_ref, d4_ref, prop_ref, nbx1_ref, nby1_ref,
                     nbx2_ref, nby2_ref, work_ref, s0_ref):
    logit = logit_ref[...]
    m = jnp.max(logit, axis=0, keepdims=True)
    e = jnp.exp(logit - m)
    ssum = jnp.sum(e, axis=0, keepdims=True)
    scores_all = e / ssum
    scores = scores_all[1:, :]

    px1 = prop_ref[0, :][None, :]
    py1 = prop_ref[1, :][None, :]
    px2 = prop_ref[2, :][None, :]
    py2 = prop_ref[3, :][None, :]
    widths = px2 - px1
    heights = py2 - py1
    ctr_x = px1 + 0.5 * widths
    ctr_y = py1 + 0.5 * heights

    dx = d4_ref[0] / 10.0
    dy = d4_ref[1] / 10.0
    dw = jnp.minimum(d4_ref[2] / 5.0, LOG_MAX)
    dh = jnp.minimum(d4_ref[3] / 5.0, LOG_MAX)

    pcx = dx * widths + ctr_x
    pcy = dy * heights + ctr_y
    pw = jnp.exp(dw) * widths
    ph = jnp.exp(dh) * heights

    x1 = jnp.clip(pcx - 0.5 * pw, 0.0, IMG_W)
    y1 = jnp.clip(pcy - 0.5 * ph, 0.0, IMG_H)
    x2 = jnp.clip(pcx + 0.5 * pw, 0.0, IMG_W)
    y2 = jnp.clip(pcy + 0.5 * ph, 0.0, IMG_H)

    cls_iota = lax.broadcasted_iota(jnp.int32, (C, NPAD), 0).astype(jnp.float32)
    offset = (cls_iota + 1.0) * OFF_STEP
    nbx1 = x1 + offset
    nby1 = y1 + offset
    nbx2 = x2 + offset
    nby2 = y2 + offset

    ws = x2 - x1
    hs = y2 - y1
    lane = lax.broadcasted_iota(jnp.int32, (C, NPAD), 1)
    valid = (scores > SCORE_T) & (ws >= MIN_SIZE) & (hs >= MIN_SIZE) \
        & (lane < N)
    work = jnp.where(valid, scores, NEG_INF)

    nbx1_ref[...] = nbx1
    nby1_ref[...] = nby1
    nbx2_ref[...] = nbx2
    nby2_ref[...] = nby2
    work_ref[...] = work
    s0_ref[...] = scores[0:1, 0:1]


def _nms_body(nbx1_ref, nby1_ref, nbx2_ref, nby2_ref, work_in_ref, s0_ref,
              out_ref, work_ref):
    work_ref[...] = work_in_ref[...]
    s0 = s0_ref[0, 0]

    lane128 = lax.broadcasted_iota(jnp.int32, (1, 128), 1)
    vec = jnp.full((1, 128), NEG_INF, dtype=jnp.float32)
    for c in range(C):
        mc = jnp.max(work_ref[c])
        vec = jnp.where(lane128 == c, mc, vec)

    sub_iota = lax.broadcasted_iota(jnp.int32, (NBLK, NSUB), 0)
    lane_iota = lax.broadcasted_iota(jnp.int32, (NBLK, NSUB), 1)
    flat_local = sub_iota * NSUB + lane_iota
    liota = lax.broadcasted_iota(jnp.int32, (1, 8), 1)

    def body(t, carry):
        vec, mc_s, cls_p, rest_s, cls_r = carry
        v = jnp.maximum(mc_s, rest_s)
        is_fb = v == NEG_INF
        cls = jnp.where(mc_s >= rest_s, cls_p, cls_r)
        cls = jnp.where(is_fb, 0, cls)

        w_c = work_ref[cls]
        eq = (w_c == v) & (~is_fb | (flat_local == 0))

        nx1 = nbx1_ref[cls]
        ny1 = nby1_ref[cls]
        nx2 = nbx2_ref[cls]
        ny2 = nby2_ref[cls]
        bx1 = jnp.sum(jnp.where(eq, nx1, 0.0))
        by1 = jnp.sum(jnp.where(eq, ny1, 0.0))
        bx2 = jnp.sum(jnp.where(eq, nx2, 0.0))
        by2 = jnp.sum(jnp.where(eq, ny2, 0.0))
        ba = (bx2 - bx1) * (by2 - by1)

        xx1 = jnp.maximum(nx1, bx1)
        yy1 = jnp.maximum(ny1, by1)
        xx2 = jnp.minimum(nx2, bx2)
        yy2 = jnp.minimum(ny2, by2)
        areas = (nx2 - nx1) * (ny2 - ny1)
        inter = jnp.maximum(xx2 - xx1, 0.0) * jnp.maximum(yy2 - yy1, 0.0)
        iou = inter / (areas + ba - inter + 1e-9)
        new_w = jnp.where((iou > NMS_T) | eq, NEG_INF, w_c)
        work_ref[cls] = new_w

        vec_m = jnp.where(lane128 == cls, NEG_INF, vec)
        rest_n = jnp.max(vec_m)
        cls_rn = jnp.min(jnp.where(vec_m == rest_n, lane128, 127))

        mc = jnp.max(new_w)
        vec = jnp.where(lane128 == cls, mc, vec)

        s_out = jnp.where(is_fb, s0, v)
        offc = (cls + 1).astype(jnp.float32) * OFF_STEP
        lbl = (cls + 1).astype(jnp.float32)
        row = jnp.where(
            liota == 0, bx1 - offc,
            jnp.where(liota == 1, by1 - offc,
                      jnp.where(liota == 2, bx2 - offc,
                                jnp.where(liota == 3, by2 - offc,
                                          jnp.where(liota == 4, s_out, lbl)))))
        out_ref[pl.ds(t, 1), :] = row
        return (vec, mc, cls, rest_n, cls_rn)

    rest0 = jnp.max(vec)
    cls_r0 = jnp.min(jnp.where(vec == rest0, lane128, 127))
    lax.fori_loop(0, NUM_DET, body,
                  (vec, jnp.float32(NEG_INF), jnp.int32(0), rest0, cls_r0))


@jax.jit
def kernel(class_logit, box_regression, proposal):
    logit_t = jnp.pad(class_logit, ((0, NPAD - N), (0, 0))).T
    d4 = jnp.pad(
        jnp.transpose(box_regression.reshape(N, NUM_CLASSES, 4)[:, 1:, :],
                      (2, 1, 0)),
        ((0, 0), (0, 0), (0, NPAD - N)))
    prop_t = jnp.pad(proposal, ((0, NPAD - N), (0, 0))).T

    big = jax.ShapeDtypeStruct((C, NPAD), jnp.float32)
    pre = pl.pallas_call(
        _precompute_body,
        out_shape=(big,) * 5 + (jax.ShapeDtypeStruct((1, 1), jnp.float32),),
        interpret=_INTERPRET,
    )(logit_t, d4, prop_t)
    nbx1, nby1, nbx2, nby2, work, s0 = pre

    shaped = [a.reshape(C, NBLK, NSUB)
              for a in (nbx1, nby1, nbx2, nby2, work)]

    out = pl.pallas_call(
        _nms_body,
        out_shape=jax.ShapeDtypeStruct((NUM_DET, 8), jnp.float32),
        scratch_shapes=[pltpu.VMEM((C, NBLK, NSUB), jnp.float32)],
        interpret=_INTERPRET,
    )(*shaped, s0)

    boxes = out[:, 0:4]
    scores = out[:, 4]
    labels = out[:, 5].astype(jnp.int32)
    return boxes, scores, labels
